# baseline (speedup 1.0000x reference)
import functools

import jax
import jax.numpy as jnp
from jax import lax
from jax.experimental import pallas as pl
from jax.experimental.pallas import tpu as pltpu
from jax.experimental.pallas import tpu_sc as plsc

B, C, H, W = 8, 96, 224, 224
NW = 32
ROWS = B * H
RPW = ROWS // NW


def _sc_body(f_hbm, bits_hbm, out_hbm, in_buf, out_buf, bits_buf):
    wid = lax.axis_index("s") * 2 + lax.axis_index("c")
    base = wid * RPW

    def row_body(g, _):
        r = base + g
        b = r // H
        h = r % H
        pltpu.sync_copy(f_hbm.at[b, :, h, :], in_buf)
        pltpu.sync_copy(bits_hbm.at[b, h, :], bits_buf)
        for j in range(W // 16):
            sl = pl.ds(j * 16, 16)

            def red_body(k, carry):
                mn, mx = carry
                v = in_buf[k, sl]
                return jnp.minimum(mn, v), jnp.maximum(mx, v)

            v0 = in_buf[0, sl]
            mn, mx = lax.fori_loop(1, C, red_body, (v0, v0))
            bits16 = jnp.clip(bits_buf[sl], 1, 8)
            lm1 = (jnp.int32(1) << bits16).astype(jnp.float32) - 1.0
            rng = mx - mn
            valid = rng > 1e-8
            su = jnp.where(valid, lm1 / jnp.where(valid, rng, 1.0), 0.0)
            sd = rng / lm1

            def ew_body(k, carry):
                v = in_buf[k, sl]
                t = (v - mn) * su + 0.5
                q = t.astype(jnp.int32).astype(jnp.float32)
                out_buf[k, sl] = q * sd + mn
                return carry

            lax.fori_loop(0, C, ew_body, 0)
        pltpu.sync_copy(out_buf, out_hbm.at[b, :, h, :])
        return _

    lax.fori_loop(0, RPW, row_body, 0)


@jax.jit
def _run_sc(features, bits_i32):
    k = functools.partial(
        pl.kernel,
        out_type=jax.ShapeDtypeStruct((B, C, H, W), jnp.float32),
        mesh=plsc.VectorSubcoreMesh(core_axis_name="c", subcore_axis_name="s"),
        scratch_types=[
            pltpu.VMEM((C, W), jnp.float32),
            pltpu.VMEM((C, W), jnp.float32),
            pltpu.VMEM((W,), jnp.int32),
        ],
    )(_sc_body)
    return k(features, bits_i32)


def kernel(features, bit_allocation):
    return _run_sc(features, bit_allocation.astype(jnp.int32))
